# baseline (speedup 1.0000x reference)
import jax
import jax.numpy as jnp
from jax import lax
from jax.experimental import pallas as pl
from jax.experimental.pallas import tpu as pltpu

N = 256
H = 512
D_IN = 512
D_OUT = 3
NCHUNK = 4
CB = H // NCHUNK


def _matmul_t(x, w):
    return lax.dot_general(x, w, (((1,), (1,)), ((), ())),
                           preferred_element_type=jnp.float32)


def _fused_kernel(x_now_hbm, W_in_hbm, h_prev_hbm,
                  Wl1_hbm, Wr1_hbm, Wl2_hbm, Wr2_hbm,
                  b_in_ref, bl1_ref, bl2_ref,
                  gamma_ref, beta_ref, W_out_ref, b_out_ref,
                  h_out_ref, out_ref,
                  x_v, Win_v, hp_v, Wl1_v, Wr1_v, Wl2_v, Wr2_v, sems):
    front = [(x_now_hbm, x_v), (W_in_hbm, Win_v), (h_prev_hbm, hp_v),
             (Wl1_hbm, Wl1_v), (Wr1_hbm, Wr1_v)]
    cps = []
    for i, (src, dst) in enumerate(front):
        cp = pltpu.make_async_copy(src, dst, sems.at[i])
        cp.start()
        cps.append(cp)
    l2 = []
    for c in range(NCHUNK):
        rows = pl.ds(c * CB, CB)
        cpl = pltpu.make_async_copy(Wl2_hbm.at[rows, :], Wl2_v.at[rows, :],
                                    sems.at[5 + 2 * c])
        cpr = pltpu.make_async_copy(Wr2_hbm.at[rows, :], Wr2_v.at[rows, :],
                                    sems.at[6 + 2 * c])
        cpl.start()
        cpr.start()
        l2.append((cpl, cpr))

    cps[0].wait()
    cps[1].wait()
    h = _matmul_t(x_v[...], Win_v[...]) + b_in_ref[...]
    cps[2].wait()
    h = h + hp_v[...]

    m1 = jnp.mean(h, axis=0, keepdims=True)
    cps[3].wait()
    a1 = _matmul_t(m1, Wl1_v[...]) + bl1_ref[...]
    cps[4].wait()
    h = jnp.maximum(_matmul_t(h, Wr1_v[...]) + a1, 0.0)

    m2 = jnp.mean(h, axis=0, keepdims=True)
    out_acc = jnp.broadcast_to(b_out_ref[...], (N, D_OUT))
    for c in range(NCHUNK):
        cols = pl.ds(c * CB, CB)
        cpl, cpr = l2[c]
        cpl.wait()
        a2 = _matmul_t(m2, Wl2_v[cols, :]) + bl2_ref[:, cols]
        cpr.wait()
        h2 = jnp.maximum(_matmul_t(h, Wr2_v[cols, :]) + a2, 0.0)
        mu = jnp.mean(h2, axis=0, keepdims=True)
        d = h2 - mu
        var = jnp.mean(d * d, axis=0, keepdims=True)
        hn = d * lax.rsqrt(var + 1e-5) * gamma_ref[:, cols] + beta_ref[:, cols]
        h_out_ref[:, cols] = hn
        out_acc = out_acc + _matmul_t(hn, W_out_ref[:, cols])
    out_ref[...] = out_acc


def kernel(h_prev, x_now, W_in, b_in, Wl1, bl1, Wr1, Wl2, bl2, Wr2, gamma, beta, W_out, b_out):
    any_spec = pl.BlockSpec(memory_space=pl.ANY)
    vmem_spec = pl.BlockSpec(memory_space=pltpu.MemorySpace.VMEM)
    h, out = pl.pallas_call(
        _fused_kernel,
        in_specs=[any_spec] * 7 + [vmem_spec] * 7,
        out_shape=(
            jax.ShapeDtypeStruct((N, H), jnp.float32),
            jax.ShapeDtypeStruct((N, D_OUT), jnp.float32),
        ),
        scratch_shapes=[
            pltpu.VMEM((N, D_IN), jnp.float32),
            pltpu.VMEM((H, D_IN), jnp.float32),
            pltpu.VMEM((N, H), jnp.float32),
            pltpu.VMEM((H, H), jnp.float32),
            pltpu.VMEM((H, H), jnp.float32),
            pltpu.VMEM((H, H), jnp.float32),
            pltpu.VMEM((H, H), jnp.float32),
            pltpu.SemaphoreType.DMA((5 + 2 * NCHUNK,)),
        ],
    )(
        x_now, W_in, h_prev, Wl1, Wr1, Wl2, Wr2,
        b_in.reshape(1, H), bl1.reshape(1, H), bl2.reshape(1, H),
        gamma.reshape(1, H), beta.reshape(1, H),
        W_out, b_out.reshape(1, D_OUT),
    )
    return h, out


# speedup vs baseline: 1.0774x; 1.0774x over previous
import jax
import jax.numpy as jnp
from jax import lax
from jax.experimental import pallas as pl
from jax.experimental.pallas import tpu as pltpu

N = 256
H = 512
D_IN = 512
D_OUT = 3
HB = H // 2


def _matmul_t(x, w):
    return lax.dot_general(x, w, (((1,), (1,)), ((), ())),
                           preferred_element_type=jnp.float32)


def _fused_kernel(x_now_hbm, W_in_hbm, h_prev_hbm,
                  Wl1_hbm, Wr1_hbm, Wl2_hbm, Wr2_hbm,
                  b_in_ref, bl1_ref, bl2_ref,
                  gamma_ref, beta_ref, W_out_ref, b_out_ref,
                  h_out_hbm, out_ref,
                  x_v, Win_v, hp_v, Wl1_v, Wr1_v, Wl2_v, Wr2_v, hn_v, sems):
    front = [(x_now_hbm, x_v), (W_in_hbm, Win_v), (h_prev_hbm, hp_v),
             (Wl1_hbm, Wl1_v), (Wr1_hbm, Wr1_v), (Wl2_hbm, Wl2_v)]
    cps = []
    for i, (src, dst) in enumerate(front):
        cp = pltpu.make_async_copy(src, dst, sems.at[i])
        cp.start()
        cps.append(cp)
    halves = []
    for c in range(2):
        rows = pl.ds(c * HB, HB)
        cp = pltpu.make_async_copy(Wr2_hbm.at[rows, :], Wr2_v.at[rows, :],
                                   sems.at[6 + c])
        cp.start()
        halves.append(cp)

    cps[0].wait()
    cps[1].wait()
    h = _matmul_t(x_v[...], Win_v[...]) + b_in_ref[...]
    cps[2].wait()
    h = h + hp_v[...]

    m1 = jnp.mean(h, axis=0, keepdims=True)
    cps[3].wait()
    a1 = _matmul_t(m1, Wl1_v[...]) + bl1_ref[...]
    cps[4].wait()
    h = jnp.maximum(_matmul_t(h, Wr1_v[...]) + a1, 0.0)

    m2 = jnp.mean(h, axis=0, keepdims=True)
    cps[5].wait()
    a2 = _matmul_t(m2, Wl2_v[...]) + bl2_ref[...]
    out_acc = jnp.broadcast_to(b_out_ref[...], (N, D_OUT))
    out_cps = []
    for c in range(2):
        cols = pl.ds(c * HB, HB)
        halves[c].wait()
        h2 = jnp.maximum(_matmul_t(h, Wr2_v[cols, :]) + a2[:, c * HB:(c + 1) * HB], 0.0)
        mu = jnp.mean(h2, axis=0, keepdims=True)
        d = h2 - mu
        var = jnp.mean(d * d, axis=0, keepdims=True)
        hn = d * lax.rsqrt(var + 1e-5) * gamma_ref[:, cols] + beta_ref[:, cols]
        hn_v[:, cols] = hn
        cp = pltpu.make_async_copy(hn_v.at[:, cols], h_out_hbm.at[:, cols],
                                   sems.at[8 + c])
        cp.start()
        out_cps.append(cp)
        out_acc = out_acc + _matmul_t(hn, W_out_ref[:, cols])
    out_ref[...] = out_acc
    for cp in out_cps:
        cp.wait()


def kernel(h_prev, x_now, W_in, b_in, Wl1, bl1, Wr1, Wl2, bl2, Wr2, gamma, beta, W_out, b_out):
    any_spec = pl.BlockSpec(memory_space=pl.ANY)
    vmem_spec = pl.BlockSpec(memory_space=pltpu.MemorySpace.VMEM)
    h, out = pl.pallas_call(
        _fused_kernel,
        in_specs=[any_spec] * 7 + [vmem_spec] * 7,
        out_specs=(any_spec, vmem_spec),
        out_shape=(
            jax.ShapeDtypeStruct((N, H), jnp.float32),
            jax.ShapeDtypeStruct((N, D_OUT), jnp.float32),
        ),
        scratch_shapes=[
            pltpu.VMEM((N, D_IN), jnp.float32),
            pltpu.VMEM((H, D_IN), jnp.float32),
            pltpu.VMEM((N, H), jnp.float32),
            pltpu.VMEM((H, H), jnp.float32),
            pltpu.VMEM((H, H), jnp.float32),
            pltpu.VMEM((H, H), jnp.float32),
            pltpu.VMEM((H, H), jnp.float32),
            pltpu.VMEM((N, H), jnp.float32),
            pltpu.SemaphoreType.DMA((10,)),
        ],
    )(
        x_now, W_in, h_prev, Wl1, Wr1, Wl2, Wr2,
        b_in.reshape(1, H), bl1.reshape(1, H), bl2.reshape(1, H),
        gamma.reshape(1, H), beta.reshape(1, H),
        W_out, b_out.reshape(1, D_OUT),
    )
    return h, out


# speedup vs baseline: 1.1953x; 1.1094x over previous
import jax
import jax.numpy as jnp
from jax import lax
from jax.experimental import pallas as pl
from jax.experimental.pallas import tpu as pltpu

N = 256
H = 512
D_IN = 512
D_OUT = 3


def _matmul_t(x, w):
    return lax.dot_general(x, w, (((1,), (1,)), ((), ())),
                           preferred_element_type=jnp.float32)


def _fused_kernel(x_now_hbm, W_in_hbm, h_prev_hbm,
                  Wl1_hbm, Wr1_hbm, Wl2_hbm, Wr2_hbm,
                  b_in_ref, bl1_ref, bl2_ref,
                  gamma_ref, beta_ref, W_out_ref, b_out_ref,
                  h_out_ref, out_ref,
                  x_v, Win_v, hp_v, Wl1_v, Wr1_v, Wl2_v, Wr2_v, sems):
    pairs = [(x_now_hbm, x_v), (W_in_hbm, Win_v), (h_prev_hbm, hp_v),
             (Wr1_hbm, Wr1_v), (Wl1_hbm, Wl1_v),
             (Wr2_hbm, Wr2_v), (Wl2_hbm, Wl2_v)]
    cps = []
    for i, (src, dst) in enumerate(pairs):
        cp = pltpu.make_async_copy(src, dst, sems.at[i])
        cp.start()
        cps.append(cp)
    (cp_x, cp_Win, cp_hp, cp_Wr1, cp_Wl1, cp_Wr2, cp_Wl2) = cps

    cp_x.wait()
    cp_Win.wait()
    h = _matmul_t(x_v[...], Win_v[...]) + b_in_ref[...]
    cp_hp.wait()
    h = h + hp_v[...]

    m1 = jnp.mean(h, axis=0, keepdims=True)
    cp_Wr1.wait()
    g1 = _matmul_t(h, Wr1_v[...])
    cp_Wl1.wait()
    a1 = _matmul_t(m1, Wl1_v[...]) + bl1_ref[...]
    h = jnp.maximum(g1 + a1, 0.0)

    m2 = jnp.mean(h, axis=0, keepdims=True)
    cp_Wr2.wait()
    g2 = _matmul_t(h, Wr2_v[...])
    cp_Wl2.wait()
    a2 = _matmul_t(m2, Wl2_v[...]) + bl2_ref[...]
    h = jnp.maximum(g2 + a2, 0.0)

    mu = jnp.mean(h, axis=0, keepdims=True)
    c = h - mu
    var = jnp.mean(c * c, axis=0, keepdims=True)
    hn = c * lax.rsqrt(var + 1e-5) * gamma_ref[...] + beta_ref[...]
    h_out_ref[...] = hn

    out_ref[...] = _matmul_t(hn, W_out_ref[...]) + b_out_ref[...]


def kernel(h_prev, x_now, W_in, b_in, Wl1, bl1, Wr1, Wl2, bl2, Wr2, gamma, beta, W_out, b_out):
    any_spec = pl.BlockSpec(memory_space=pl.ANY)
    vmem_spec = pl.BlockSpec(memory_space=pltpu.MemorySpace.VMEM)
    h, out = pl.pallas_call(
        _fused_kernel,
        in_specs=[any_spec] * 7 + [vmem_spec] * 7,
        out_shape=(
            jax.ShapeDtypeStruct((N, H), jnp.float32),
            jax.ShapeDtypeStruct((N, D_OUT), jnp.float32),
        ),
        scratch_shapes=[
            pltpu.VMEM((N, D_IN), jnp.float32),
            pltpu.VMEM((H, D_IN), jnp.float32),
            pltpu.VMEM((N, H), jnp.float32),
            pltpu.VMEM((H, H), jnp.float32),
            pltpu.VMEM((H, H), jnp.float32),
            pltpu.VMEM((H, H), jnp.float32),
            pltpu.VMEM((H, H), jnp.float32),
            pltpu.SemaphoreType.DMA((7,)),
        ],
    )(
        x_now, W_in, h_prev, Wl1, Wr1, Wl2, Wr2,
        b_in.reshape(1, H), bl1.reshape(1, H), bl2.reshape(1, H),
        gamma.reshape(1, H), beta.reshape(1, H),
        W_out, b_out.reshape(1, D_OUT),
    )
    return h, out
